# baseline (speedup 1.0000x reference)
import functools

import jax
import jax.numpy as jnp
from jax import lax
from jax.experimental import pallas as pl
from jax.experimental.pallas import tpu as pltpu
from jax.experimental.pallas import tpu_sc as plsc

_NUM_BLUEPRINTS = 1_000_000
_NULL_IDX = _NUM_BLUEPRINTS
_D = 32
_L = 16
_BCHUNK = 128
_NC = 2
_NS = 16
_NW = _NC * _NS


def _make_kernel(s, b):
    b_per_w = b // _NW
    chunks = b_per_w // _BCHUNK
    s_pairs = s // 2
    s_odd = s % 2

    mesh = plsc.VectorSubcoreMesh(
        core_axis_name="c", subcore_axis_name="s",
        num_cores=_NC, num_subcores=_NS)

    @functools.partial(
        pl.kernel,
        out_type=jax.ShapeDtypeStruct((s, _D, b), jnp.float32),
        mesh=mesh,
        compiler_params=pltpu.CompilerParams(
            use_tc_tiling_on_sc=False, needs_layout_passes=False),
        scratch_types=[
            pltpu.VMEM((2, s, _BCHUNK), jnp.int32),
            pltpu.VMEM((2, _BCHUNK, _D), jnp.float32),
            pltpu.VMEM((2, _D, _BCHUNK), jnp.float32),
            pltpu.SemaphoreType.DMA,
            pltpu.SemaphoreType.DMA,
            pltpu.SemaphoreType.DMA,
            pltpu.SemaphoreType.DMA,
        ],
    )
    def k(idx_hbm, table_hbm, out_hbm, idx_v, rows_v, tout_v, g0, g1, t0, t1):
        wid = lax.axis_index("s") * _NC + lax.axis_index("c")
        gsems = (g0, g1)
        ssems = (t0, t1)
        lane = lax.iota(jnp.int32, _L)

        def remap(islot):
            def body(j, carry):
                for l in range(_BCHUNK // _L):
                    v = idx_v[islot, j, pl.ds(l * _L, _L)]
                    v = jnp.where(v < 0, jnp.int32(_NULL_IDX), v)
                    idx_v[islot, j, pl.ds(l * _L, _L)] = v
                return carry
            lax.fori_loop(0, s, body, 0)

        def gather(islot, sidx, slot):
            return pltpu.async_copy(
                table_hbm.at[idx_v.at[islot, sidx]],
                rows_v.at[slot], gsems[slot])

        def transpose(slot):
            def body(d, carry):
                col = jnp.full((_L,), d, jnp.int32)
                for g in range(_BCHUNK // _L):
                    v = plsc.load_gather(rows_v.at[slot],
                                         [lane + (g * _L), col])
                    tout_v[slot, d, pl.ds(g * _L, _L)] = v
                return carry
            lax.fori_loop(0, _D, body, 0)

        def store(sidx, b0, slot):
            return pltpu.async_copy(
                tout_v.at[slot],
                out_hbm.at[sidx, :, pl.ds(b0, _BCHUNK)],
                ssems[slot])

        for c in range(chunks):
            b0 = wid * b_per_w + c * _BCHUNK
            islot = c % 2
            pltpu.sync_copy(idx_hbm.at[:, pl.ds(b0, _BCHUNK)],
                            idx_v.at[islot])
            remap(islot)

            def pair_body(p, carry2, islot=islot, b0=b0):
                sa = 2 * p
                sb = 2 * p + 1
                cpa = gather(islot, sa, 0)
                cpb = gather(islot, sb, 1)
                cpa.wait()
                transpose(0)
                sta = store(sa, b0, 0)
                cpb.wait()
                transpose(1)
                stb = store(sb, b0, 1)
                sta.wait()
                stb.wait()
                return carry2

            lax.fori_loop(0, s_pairs, pair_body, 0)
            if s_odd:
                cpz = gather(islot, s - 1, 0)
                cpz.wait()
                transpose(0)
                store(s - 1, b0, 0).wait()

    return k


def kernel(blueprint_indices, embedding_weight):
    b, s = blueprint_indices.shape
    idx_t = jnp.transpose(blueprint_indices.astype(jnp.int32))
    out_t = _make_kernel(s, b)(idx_t, embedding_weight)
    return jnp.transpose(out_t, (2, 0, 1))


# speedup vs baseline: 1.2544x; 1.2544x over previous
import functools

import jax
import jax.numpy as jnp
from jax import lax
from jax.experimental import pallas as pl
from jax.experimental.pallas import tpu as pltpu
from jax.experimental.pallas import tpu_sc as plsc

_NUM_BLUEPRINTS = 1_000_000
_NULL_IDX = _NUM_BLUEPRINTS
_D = 32
_L = 16
_BCHUNK = 128
_RING = 4
_NC = 2
_NS = 16
_NW = _NC * _NS


def _make_kernel(s, b):
    b_per_w = b // _NW
    chunks = b_per_w // _BCHUNK
    rounds = s // _RING
    tail = s % _RING

    mesh = plsc.VectorSubcoreMesh(
        core_axis_name="c", subcore_axis_name="s",
        num_cores=_NC, num_subcores=_NS)

    @functools.partial(
        pl.kernel,
        out_type=jax.ShapeDtypeStruct((s, _D, b), jnp.float32),
        mesh=mesh,
        compiler_params=pltpu.CompilerParams(
            use_tc_tiling_on_sc=False, needs_layout_passes=False),
        scratch_types=[
            pltpu.VMEM((2, s, _BCHUNK), jnp.int32),
            pltpu.VMEM((_RING, _BCHUNK, _D), jnp.float32),
            pltpu.VMEM((_RING, _D, _BCHUNK), jnp.float32),
        ]
        + [pltpu.SemaphoreType.DMA] * (2 * _RING),
    )
    def k(idx_hbm, table_hbm, out_hbm, idx_v, rows_v, tout_v, *sems):
        gsems = sems[:_RING]
        ssems = sems[_RING:]
        wid = lax.axis_index("s") * _NC + lax.axis_index("c")
        lane = lax.iota(jnp.int32, _L)

        def remap(islot):
            def body(j, carry):
                for l in range(_BCHUNK // _L):
                    v = idx_v[islot, j, pl.ds(l * _L, _L)]
                    v = jnp.where(v < 0, jnp.int32(_NULL_IDX), v)
                    idx_v[islot, j, pl.ds(l * _L, _L)] = v
                return carry
            lax.fori_loop(0, s, body, 0)

        def gather(islot, sidx, slot):
            return pltpu.async_copy(
                table_hbm.at[idx_v.at[islot, sidx]],
                rows_v.at[slot], gsems[slot])

        def transpose(slot):
            def body(d, carry):
                col = jnp.full((_L,), d, jnp.int32)
                vs = [plsc.load_gather(rows_v.at[slot],
                                       [lane + (g * _L), col])
                      for g in range(_BCHUNK // _L)]
                for g, v in enumerate(vs):
                    tout_v[slot, d, pl.ds(g * _L, _L)] = v
                return carry
            lax.fori_loop(0, _D, body, 0)

        def store(sidx, b0, slot):
            return pltpu.async_copy(
                tout_v.at[slot],
                out_hbm.at[sidx, :, pl.ds(b0, _BCHUNK)],
                ssems[slot])

        def st_wait(sidx, b0, slot):
            pltpu.make_async_copy(
                tout_v.at[slot],
                out_hbm.at[sidx, :, pl.ds(b0, _BCHUNK)],
                ssems[slot]).wait()

        def g_wait(islot, sidx, slot):
            pltpu.make_async_copy(
                table_hbm.at[idx_v.at[islot, sidx]],
                rows_v.at[slot], gsems[slot]).wait()

        for c in range(chunks):
            b0 = wid * b_per_w + c * _BCHUNK
            islot = c % 2
            pltpu.sync_copy(idx_hbm.at[:, pl.ds(b0, _BCHUNK)],
                            idx_v.at[islot])
            remap(islot)

            for u in range(_RING):
                gather(islot, u, u)

            def round_body(r, carry, islot=islot, b0=b0):
                base = r * _RING
                for u in range(_RING):
                    sidx = base + u
                    g_wait(islot, sidx, u)
                    st_wait(sidx - _RING, b0, u)
                    transpose(u)
                    store(sidx, b0, u)

                    @pl.when(sidx + _RING < s)
                    def _():
                        gather(islot, sidx + _RING, u)
                return carry

            for u in range(_RING):
                g_wait(islot, u, u)
                transpose(u)
                store(u, b0, u)
                gather(islot, u + _RING, u)

            lax.fori_loop(1, rounds, round_body, 0)

            for u in range(tail):
                sidx = rounds * _RING + u
                g_wait(islot, sidx, u)
                st_wait(sidx - _RING, b0, u)
                transpose(u)
                store(sidx, b0, u)

            for u in range(tail, _RING):
                st_wait((rounds - 1) * _RING + u, b0, u)
            for u in range(tail):
                st_wait(rounds * _RING + u, b0, u)

    return k


def kernel(blueprint_indices, embedding_weight):
    b, s = blueprint_indices.shape
    idx_t = jnp.transpose(blueprint_indices.astype(jnp.int32))
    out_t = _make_kernel(s, b)(idx_t, embedding_weight)
    return jnp.transpose(out_t, (2, 0, 1))
